# baseline (speedup 1.0000x reference)
import functools

import jax
import jax.numpy as jnp
from jax.experimental import pallas as pl
from jax.experimental.pallas import tpu as pltpu

_EPS = 1e-5
_VMEM = 60 * 1024 * 1024


def _pcbactiv_kernel(x_ref, m_ref, w_ref, g_ref, b_ref, h_ref, nm_ref,
                     xp3_s, box_s, hole_s, conv_s, psum_s, pssq_s,
                     *, C, O, H, W, N):
    p = pl.program_id(0)
    n = pl.program_id(1)
    t = pl.program_id(2)
    HW = H * W
    H2 = H // 2
    HW2 = HW // 2
    F = (H + 2) * W
    R = N * HW

    @pl.when(jnp.logical_and(p == 0, t == 0))
    def _build():
        xm = (x_ref[0] * m_ref[0]).astype(jnp.bfloat16).reshape(C, HW)
        zrow = jnp.zeros((C, W), jnp.bfloat16)
        xp = jnp.concatenate([zrow, xm, zrow], axis=1)
        col = jax.lax.broadcasted_iota(jnp.int32, (1, F), 1) % W
        zc = jnp.zeros((C, 1), jnp.bfloat16)
        a0 = jnp.where(col == 0, jnp.bfloat16(0),
                       jnp.concatenate([zc, xp[:, :-1]], axis=1))
        a2 = jnp.where(col == W - 1, jnp.bfloat16(0),
                       jnp.concatenate([xp[:, 1:], zc], axis=1))
        xp3_s[...] = jnp.concatenate([a0, xp, a2], axis=0)

        mcs = jnp.sum(m_ref[0], axis=0)
        zr = jnp.zeros((1, W), jnp.float32)
        mh = jnp.concatenate([zr, mcs, zr], axis=0)
        vert = mh[:H] + mh[1:H + 1] + mh[2:H + 2]
        zc1 = jnp.zeros((H, 1), jnp.float32)
        box2 = (jnp.concatenate([zc1, vert[:, :-1]], axis=1) + vert
                + jnp.concatenate([vert[:, 1:], zc1], axis=1))
        box_s[...] = box2
        hole_s[n] = (box2 > 0.0).astype(jnp.bfloat16)

    @pl.when(p == 0)
    def _conv_half():
        @pl.when(jnp.logical_and(n == 0, t == 0))
        def _init():
            psum_s[...] = jnp.zeros_like(psum_s)
            pssq_s[...] = jnp.zeros_like(pssq_s)

        rhs = xp3_s[:, pl.ds(t * HW2, HW2 + 2 * W)]
        y = jax.lax.dot_general(w_ref[...], rhs,
                                (((1,), (0,)), ((), ())),
                                preferred_element_type=jnp.float32)
        conv = (y[:O, :HW2] + y[O:2 * O, W:W + HW2]
                + y[2 * O:, 2 * W:2 * W + HW2])

        box = box_s[pl.ds(t * H2, H2), :].reshape(1, HW2)
        holes = box == 0.0
        out = jnp.where(holes, 0.0, conv / jnp.where(holes, 1.0, box))

        conv_s[n, :, pl.ds(t * HW2, HW2)] = out.astype(jnp.bfloat16)
        psum_s[...] += jnp.sum(out, axis=1, keepdims=True)
        pssq_s[...] += jnp.sum(out * out, axis=1, keepdims=True)

    @pl.when(p == 1)
    def _bn_relu():
        mean = psum_s[...] / R
        var = jnp.maximum(pssq_s[...] / R - mean * mean, 0.0)
        scale = g_ref[...] * jax.lax.rsqrt(var + _EPS)
        shift = b_ref[...] - mean * scale
        v = jnp.maximum(
            conv_s[n, :, pl.ds(t * HW2, HW2)].astype(jnp.float32) * scale
            + shift, 0.0)
        h_ref[0] = v.reshape(O, H2, W)
        nm_ref[0] = jnp.broadcast_to(
            hole_s[n, pl.ds(t * H2, H2), :].astype(jnp.float32)[None],
            (O, H2, W))


def kernel(x, mask, weight, gamma, beta):
    N, C, H, W = x.shape
    O = weight.shape[0]
    HW = H * W
    F = (H + 2) * W
    wt = (weight.transpose(2, 0, 3, 1).reshape(3 * O, 3 * C)
          .astype(jnp.bfloat16))

    h, new_mask = pl.pallas_call(
        functools.partial(_pcbactiv_kernel, C=C, O=O, H=H, W=W, N=N),
        out_shape=(jax.ShapeDtypeStruct((N, O, H, W), jnp.float32),
                   jax.ShapeDtypeStruct((N, O, H, W), jnp.float32)),
        grid=(2, N, 2),
        in_specs=[
            pl.BlockSpec((1, C, H, W),
                         lambda p, n, t: (jnp.where(p == 0, n, N - 1),
                                          0, 0, 0)),
            pl.BlockSpec((1, C, H, W),
                         lambda p, n, t: (jnp.where(p == 0, n, N - 1),
                                          0, 0, 0)),
            pl.BlockSpec((3 * O, 3 * C), lambda p, n, t: (0, 0)),
            pl.BlockSpec((O, 1), lambda p, n, t: (0, 0)),
            pl.BlockSpec((O, 1), lambda p, n, t: (0, 0)),
        ],
        out_specs=(
            pl.BlockSpec((1, O, H // 2, W),
                         lambda p, n, t: (jnp.where(p == 0, 0, n), 0,
                                          jnp.where(p == 0, 0, t), 0)),
            pl.BlockSpec((1, O, H // 2, W),
                         lambda p, n, t: (jnp.where(p == 0, 0, n), 0,
                                          jnp.where(p == 0, 0, t), 0)),
        ),
        scratch_shapes=[
            pltpu.VMEM((3 * C, F), jnp.bfloat16),
            pltpu.VMEM((H, W), jnp.float32),
            pltpu.VMEM((N, H, W), jnp.bfloat16),
            pltpu.VMEM((N, O, HW), jnp.bfloat16),
            pltpu.VMEM((O, 1), jnp.float32),
            pltpu.VMEM((O, 1), jnp.float32),
        ],
        compiler_params=pltpu.CompilerParams(
            dimension_semantics=("arbitrary", "arbitrary", "arbitrary"),
            vmem_limit_bytes=_VMEM),
    )(x, mask, wt, gamma.reshape(O, 1).astype(jnp.float32),
      beta.reshape(O, 1).astype(jnp.float32))

    return h, new_mask


# speedup vs baseline: 1.1239x; 1.1239x over previous
import functools

import jax
import jax.numpy as jnp
from jax.experimental import pallas as pl
from jax.experimental.pallas import tpu as pltpu

_EPS = 1e-5
_VMEM = 56 * 1024 * 1024


def _pconv_kernel(x_ref, m_ref, w_ref, conv_ref, psum_ref, pssq_ref,
                  hole_ref, *, C, O, H, W):
    HW = H * W
    F = (H + 2) * W
    x = x_ref[0]
    m = m_ref[0]
    xm = (x * m).astype(jnp.bfloat16).reshape(C, HW)

    zrow = jnp.zeros((C, W), jnp.bfloat16)
    xp = jnp.concatenate([zrow, xm, zrow], axis=1)

    col = jax.lax.broadcasted_iota(jnp.int32, (1, F), 1) % W
    zc = jnp.zeros((C, 1), jnp.bfloat16)
    a0 = jnp.where(col == 0, jnp.bfloat16(0),
                   jnp.concatenate([zc, xp[:, :-1]], axis=1))
    a2 = jnp.where(col == W - 1, jnp.bfloat16(0),
                   jnp.concatenate([xp[:, 1:], zc], axis=1))
    xp3 = jnp.concatenate([a0, xp, a2], axis=0)

    y = jax.lax.dot_general(w_ref[...], xp3,
                            (((1,), (0,)), ((), ())),
                            preferred_element_type=jnp.float32)
    conv = (y[:O, :HW] + y[O:2 * O, W:W + HW]
            + y[2 * O:, 2 * W:2 * W + HW])

    mcs = jnp.sum(m, axis=0)
    zr = jnp.zeros((1, W), jnp.float32)
    mh = jnp.concatenate([zr, mcs, zr], axis=0)
    vert = mh[:H] + mh[1:H + 1] + mh[2:H + 2]
    zc1 = jnp.zeros((H, 1), jnp.float32)
    box2 = (jnp.concatenate([zc1, vert[:, :-1]], axis=1) + vert
            + jnp.concatenate([vert[:, 1:], zc1], axis=1))
    box = box2.reshape(1, HW)

    holes = box == 0.0
    out = jnp.where(holes, 0.0, conv / jnp.where(holes, 1.0, box))

    conv_ref[0] = out.astype(jnp.bfloat16)
    psum_ref[0] = jnp.sum(out, axis=1, keepdims=True)
    pssq_ref[0] = jnp.sum(out * out, axis=1, keepdims=True)
    hole_ref[0, 0] = (box2 > 0.0).astype(jnp.float32)


def _bn_relu_kernel(conv_ref, hole_ref, psum_ref, pssq_ref, g_ref, b_ref,
                    out_ref, nm_ref, *, O, HB, W, R):
    mean = jnp.sum(psum_ref[...], axis=0) / R
    var = jnp.maximum(jnp.sum(pssq_ref[...], axis=0) / R - mean * mean, 0.0)
    scale = g_ref[...] * jax.lax.rsqrt(var + _EPS)
    shift = b_ref[...] - mean * scale
    v = jnp.maximum(
        conv_ref[0].astype(jnp.float32) * scale + shift, 0.0)
    out_ref[0] = v.reshape(O, HB, W)
    nm_ref[0] = jnp.broadcast_to(hole_ref[0], (O, HB, W))


def kernel(x, mask, weight, gamma, beta):
    N, C, H, W = x.shape
    O = weight.shape[0]
    HW = H * W
    wt = (weight.transpose(2, 0, 3, 1).reshape(3 * O, 3 * C)
          .astype(jnp.bfloat16))

    conv, psum, pssq, hole = pl.pallas_call(
        functools.partial(_pconv_kernel, C=C, O=O, H=H, W=W),
        out_shape=(jax.ShapeDtypeStruct((N, O, HW), jnp.bfloat16),
                   jax.ShapeDtypeStruct((N, O, 1), jnp.float32),
                   jax.ShapeDtypeStruct((N, O, 1), jnp.float32),
                   jax.ShapeDtypeStruct((N, 1, H, W), jnp.float32)),
        grid=(N,),
        in_specs=[pl.BlockSpec((1, C, H, W), lambda n: (n, 0, 0, 0)),
                  pl.BlockSpec((1, C, H, W), lambda n: (n, 0, 0, 0)),
                  pl.BlockSpec((3 * O, 3 * C), lambda n: (0, 0))],
        out_specs=(pl.BlockSpec((1, O, HW), lambda n: (n, 0, 0)),
                   pl.BlockSpec((1, O, 1), lambda n: (n, 0, 0)),
                   pl.BlockSpec((1, O, 1), lambda n: (n, 0, 0)),
                   pl.BlockSpec((1, 1, H, W), lambda n: (n, 0, 0, 0))),
        compiler_params=pltpu.CompilerParams(
            dimension_semantics=("parallel",),
            vmem_limit_bytes=_VMEM),
    )(x, mask, wt)

    T = 2 if H % 2 == 0 else 1
    HB = H // T
    h, new_mask = pl.pallas_call(
        functools.partial(_bn_relu_kernel, O=O, HB=HB, W=W, R=N * HW),
        out_shape=(jax.ShapeDtypeStruct((N, O, H, W), jnp.float32),
                   jax.ShapeDtypeStruct((N, O, H, W), jnp.float32)),
        grid=(N, T),
        in_specs=[pl.BlockSpec((1, O, HW // T), lambda n, t: (n, 0, t)),
                  pl.BlockSpec((1, 1, HB, W), lambda n, t: (n, 0, t, 0)),
                  pl.BlockSpec((N, O, 1), lambda n, t: (0, 0, 0)),
                  pl.BlockSpec((N, O, 1), lambda n, t: (0, 0, 0)),
                  pl.BlockSpec((O, 1), lambda n, t: (0, 0)),
                  pl.BlockSpec((O, 1), lambda n, t: (0, 0))],
        out_specs=(pl.BlockSpec((1, O, HB, W), lambda n, t: (n, 0, t, 0)),
                   pl.BlockSpec((1, O, HB, W), lambda n, t: (n, 0, t, 0))),
        compiler_params=pltpu.CompilerParams(
            dimension_semantics=("parallel", "parallel"),
            vmem_limit_bytes=_VMEM),
    )(conv, hole, psum, pssq, gamma.reshape(O, 1).astype(jnp.float32),
      beta.reshape(O, 1).astype(jnp.float32))

    return h, new_mask


# speedup vs baseline: 1.1791x; 1.0491x over previous
import functools

import jax
import jax.numpy as jnp
from jax.experimental import pallas as pl
from jax.experimental.pallas import tpu as pltpu

_EPS = 1e-5
_VMEM = 56 * 1024 * 1024


def _pconv_kernel(x_ref, m_ref, w_ref, conv_ref, psum_ref, pssq_ref,
                  hole_ref, *, C, O, H, W):
    HW = H * W
    F = (H + 2) * W
    x = x_ref[0]
    m = m_ref[0]
    xm = (x * m).astype(jnp.bfloat16).reshape(C, HW)

    zrow = jnp.zeros((C, W), jnp.bfloat16)
    xp = jnp.concatenate([zrow, xm, zrow], axis=1)

    col = jax.lax.broadcasted_iota(jnp.int32, (1, F), 1) % W
    zc = jnp.zeros((C, 1), jnp.bfloat16)
    a0 = jnp.where(col == 0, jnp.bfloat16(0),
                   jnp.concatenate([zc, xp[:, :-1]], axis=1))
    a2 = jnp.where(col == W - 1, jnp.bfloat16(0),
                   jnp.concatenate([xp[:, 1:], zc], axis=1))
    xp3 = jnp.concatenate([a0, xp, a2], axis=0)

    y = jax.lax.dot_general(w_ref[...], xp3,
                            (((1,), (0,)), ((), ())),
                            preferred_element_type=jnp.float32)
    conv = (y[:O, :HW] + y[O:2 * O, W:W + HW]
            + y[2 * O:, 2 * W:2 * W + HW])

    mcs = jnp.sum(m, axis=0)
    zr = jnp.zeros((1, W), jnp.float32)
    mh = jnp.concatenate([zr, mcs, zr], axis=0)
    vert = mh[:H] + mh[1:H + 1] + mh[2:H + 2]
    zc1 = jnp.zeros((H, 1), jnp.float32)
    box2 = (jnp.concatenate([zc1, vert[:, :-1]], axis=1) + vert
            + jnp.concatenate([vert[:, 1:], zc1], axis=1))
    box = box2.reshape(1, HW)

    holes = box == 0.0
    out = jnp.where(holes, 0.0, conv / jnp.where(holes, 1.0, box))

    conv_ref[0] = out.astype(jnp.bfloat16)
    psum_ref[0] = jnp.sum(out, axis=1, keepdims=True)
    pssq_ref[0] = jnp.sum(out * out, axis=1, keepdims=True)
    hole_ref[0, 0] = (box2 > 0.0).astype(jnp.float32)


def _bn_relu_kernel(conv_ref, hole_ref, psum_ref, pssq_ref, g_ref, b_ref,
                    out_ref, nm_ref, *, O, HB, W, R):
    mean = jnp.sum(psum_ref[...], axis=0) / R
    var = jnp.maximum(jnp.sum(pssq_ref[...], axis=0) / R - mean * mean, 0.0)
    scale = g_ref[...] * jax.lax.rsqrt(var + _EPS)
    shift = b_ref[...] - mean * scale
    v = jnp.maximum(
        conv_ref[0].astype(jnp.float32) * scale + shift, 0.0)
    out_ref[0] = v.reshape(O, HB, W)
    nm_ref[0] = jnp.broadcast_to(hole_ref[0], (O, HB, W))


def kernel(x, mask, weight, gamma, beta):
    N, C, H, W = x.shape
    O = weight.shape[0]
    HW = H * W
    wt = (weight.transpose(2, 0, 3, 1).reshape(3 * O, 3 * C)
          .astype(jnp.bfloat16))

    conv, psum, pssq, hole = pl.pallas_call(
        functools.partial(_pconv_kernel, C=C, O=O, H=H, W=W),
        out_shape=(jax.ShapeDtypeStruct((N, O, HW), jnp.bfloat16),
                   jax.ShapeDtypeStruct((N, O, 1), jnp.float32),
                   jax.ShapeDtypeStruct((N, O, 1), jnp.float32),
                   jax.ShapeDtypeStruct((N, 1, H, W), jnp.float32)),
        grid=(N,),
        in_specs=[pl.BlockSpec((1, C, H, W), lambda n: (n, 0, 0, 0)),
                  pl.BlockSpec((1, C, H, W), lambda n: (n, 0, 0, 0)),
                  pl.BlockSpec((3 * O, 3 * C), lambda n: (0, 0))],
        out_specs=(pl.BlockSpec((1, O, HW), lambda n: (n, 0, 0)),
                   pl.BlockSpec((1, O, 1), lambda n: (n, 0, 0)),
                   pl.BlockSpec((1, O, 1), lambda n: (n, 0, 0)),
                   pl.BlockSpec((1, 1, H, W), lambda n: (n, 0, 0, 0))),
        compiler_params=pltpu.CompilerParams(
            dimension_semantics=("parallel",),
            vmem_limit_bytes=_VMEM),
    )(x, mask, wt)

    T = 1
    HB = H // T
    h, new_mask = pl.pallas_call(
        functools.partial(_bn_relu_kernel, O=O, HB=HB, W=W, R=N * HW),
        out_shape=(jax.ShapeDtypeStruct((N, O, H, W), jnp.float32),
                   jax.ShapeDtypeStruct((N, O, H, W), jnp.float32)),
        grid=(N, T),
        in_specs=[pl.BlockSpec((1, O, HW // T), lambda n, t: (n, 0, t)),
                  pl.BlockSpec((1, 1, HB, W), lambda n, t: (n, 0, t, 0)),
                  pl.BlockSpec((N, O, 1), lambda n, t: (0, 0, 0)),
                  pl.BlockSpec((N, O, 1), lambda n, t: (0, 0, 0)),
                  pl.BlockSpec((O, 1), lambda n, t: (0, 0)),
                  pl.BlockSpec((O, 1), lambda n, t: (0, 0))],
        out_specs=(pl.BlockSpec((1, O, HB, W), lambda n, t: (n, 0, t, 0)),
                   pl.BlockSpec((1, O, HB, W), lambda n, t: (n, 0, t, 0))),
        compiler_params=pltpu.CompilerParams(
            dimension_semantics=("parallel", "parallel"),
            vmem_limit_bytes=_VMEM),
    )(conv, hole, psum, pssq, gamma.reshape(O, 1).astype(jnp.float32),
      beta.reshape(O, 1).astype(jnp.float32))

    return h, new_mask
